# Initial kernel scaffold; baseline (speedup 1.0000x reference)
import jax
import jax.numpy as jnp
from jax.experimental import pallas as pl


def kernel(input, input_lengths, table):
    raise NotImplementedError("write your pallas kernel here")


# speedup vs baseline: 4.1425x; 4.1425x over previous
import functools
import jax
import jax.numpy as jnp
from jax import lax
from jax.experimental import pallas as pl
from jax.experimental.pallas import tpu as pltpu
from jax.experimental.pallas import tpu_sc as plsc

VOCAB = 100000
EMBED_DIM = 64
BATCH = 4096
SEQ = 200

TOT = BATCH * SEQ
NW = 32
PER_W = TOT // NW
IDX_W = 128
CHUNK = 1024
K = CHUNK // IDX_W
NCH = PER_W // CHUNK

_mesh = plsc.VectorSubcoreMesh(core_axis_name="c", subcore_axis_name="s")


@functools.partial(
    pl.kernel,
    mesh=_mesh,
    out_type=jax.ShapeDtypeStruct((TOT, EMBED_DIM), jnp.float32),
    compiler_params=pltpu.CompilerParams(use_tc_tiling_on_sc=False),
    scratch_types=[
        pltpu.VMEM((K, IDX_W), jnp.int32),
        pltpu.VMEM((CHUNK, EMBED_DIM), jnp.float32),
        pltpu.SemaphoreType.DMA,
    ],
)
def _embed_gather(table_hbm, idx_hbm, out_hbm, idx_v, rows_v, sem):
    wid = lax.axis_index("s") * 2 + lax.axis_index("c")
    chunk0 = wid * NCH

    def body(c, carry):
        row_off = (chunk0 + c) * CHUNK
        pltpu.sync_copy(idx_hbm.at[pl.ds((chunk0 + c) * K, K)], idx_v)
        for j in range(K):
            pltpu.async_copy(
                table_hbm.at[idx_v.at[j]],
                rows_v.at[pl.ds(j * IDX_W, IDX_W)],
                sem,
            )
        for j in range(K):
            pltpu.make_async_copy(
                table_hbm.at[idx_v.at[j]],
                rows_v.at[pl.ds(j * IDX_W, IDX_W)],
                sem,
            ).wait()
        pltpu.sync_copy(rows_v, out_hbm.at[pl.ds(row_off, CHUNK)])
        return carry

    lax.fori_loop(0, NCH, body, 0)


def kernel(input, input_lengths, table):
    idx = jnp.asarray(input, jnp.int32).reshape(TOT // IDX_W, IDX_W)
    out = _embed_gather(table, idx)
    return (out.reshape(BATCH, SEQ, EMBED_DIM), input_lengths)


# speedup vs baseline: 4.2601x; 1.0284x over previous
import functools
import jax
import jax.numpy as jnp
from jax import lax
from jax.experimental import pallas as pl
from jax.experimental.pallas import tpu as pltpu
from jax.experimental.pallas import tpu_sc as plsc

VOCAB = 100000
EMBED_DIM = 64
BATCH = 4096
SEQ = 200

TOT = BATCH * SEQ
NW = 32
PER_W = TOT // NW
IDX_W = 128
IDX_ROWS = PER_W // IDX_W
CHUNK = 512
K = CHUNK // IDX_W
NCH = PER_W // CHUNK
NBUF = 2

_mesh = plsc.VectorSubcoreMesh(core_axis_name="c", subcore_axis_name="s")


@functools.partial(
    pl.kernel,
    mesh=_mesh,
    out_type=jax.ShapeDtypeStruct((TOT, EMBED_DIM), jnp.float32),
    compiler_params=pltpu.CompilerParams(use_tc_tiling_on_sc=False),
    scratch_types=[
        pltpu.VMEM((IDX_ROWS, IDX_W), jnp.int32),
        pltpu.VMEM((NBUF, CHUNK, EMBED_DIM), jnp.float32),
        pltpu.SemaphoreType.DMA,
        pltpu.SemaphoreType.DMA,
        pltpu.SemaphoreType.DMA,
    ],
)
def _embed_gather(table_hbm, idx_hbm, out_hbm, idx_all, rows_v, sem_g,
                  sem_w0, sem_w1):
    wid = lax.axis_index("s") * 2 + lax.axis_index("c")
    row0 = wid * PER_W
    sem_w = (sem_w0, sem_w1)

    pltpu.sync_copy(idx_hbm.at[pl.ds(wid * IDX_ROWS, IDX_ROWS)], idx_all)

    def do_chunk(c, b):
        buf = rows_v.at[b]
        for j in range(K):
            pltpu.async_copy(
                table_hbm.at[idx_all.at[c * K + j]],
                buf.at[pl.ds(j * IDX_W, IDX_W)],
                sem_g,
            )
        for j in range(K):
            pltpu.make_async_copy(
                table_hbm.at[idx_all.at[c * K + j]],
                buf.at[pl.ds(j * IDX_W, IDX_W)],
                sem_g,
            ).wait()
        pltpu.async_copy(buf, out_hbm.at[pl.ds(row0 + c * CHUNK, CHUNK)],
                         sem_w[b])

    def wait_write(c, b):
        pltpu.make_async_copy(
            rows_v.at[b],
            out_hbm.at[pl.ds(row0 + c * CHUNK, CHUNK)],
            sem_w[b],
        ).wait()

    for b in range(NBUF):
        do_chunk(b, b)

    @pl.loop(NBUF, NCH, step=NBUF)
    def _(cc):
        for b in range(NBUF):
            wait_write(cc + b - NBUF, b)
            do_chunk(cc + b, b)

    for b in range(NBUF):
        wait_write(NCH - NBUF + b, b)


def kernel(input, input_lengths, table):
    idx = jnp.asarray(input, jnp.int32).reshape(TOT // IDX_W, IDX_W)
    out = _embed_gather(table, idx)
    return (out.reshape(BATCH, SEQ, EMBED_DIM), input_lengths)


# speedup vs baseline: 7.0617x; 1.6576x over previous
import functools
import jax
import jax.numpy as jnp
from jax import lax
from jax.experimental import pallas as pl
from jax.experimental.pallas import tpu as pltpu
from jax.experimental.pallas import tpu_sc as plsc

VOCAB = 100000
EMBED_DIM = 64
BATCH = 4096
SEQ = 200

PAD_DIM = 128
TOT = BATCH * SEQ
NW = 32
PER_W = TOT // NW
IDX_W = 128
IDX_ROWS = PER_W // IDX_W
CHUNK = 256
K = CHUNK // IDX_W
NCH = PER_W // CHUNK
NBUF = 2

_mesh = plsc.VectorSubcoreMesh(core_axis_name="c", subcore_axis_name="s")


@functools.partial(
    pl.kernel,
    mesh=_mesh,
    out_type=jax.ShapeDtypeStruct((TOT, PAD_DIM), jnp.float32),
    compiler_params=pltpu.CompilerParams(use_tc_tiling_on_sc=False),
    scratch_types=[
        pltpu.VMEM((IDX_ROWS, IDX_W), jnp.int32),
        pltpu.VMEM((NBUF, CHUNK, EMBED_DIM), jnp.float32),
        pltpu.SemaphoreType.DMA,
        pltpu.SemaphoreType.DMA,
        pltpu.SemaphoreType.DMA,
    ],
)
def _embed_gather(table_hbm, idx_hbm, out_hbm, idx_all, rows_v, sem_g,
                  sem_w0, sem_w1):
    wid = lax.axis_index("s") * 2 + lax.axis_index("c")
    row0 = wid * PER_W
    sem_w = (sem_w0, sem_w1)

    pltpu.sync_copy(idx_hbm.at[pl.ds(wid * IDX_ROWS, IDX_ROWS)], idx_all)

    def do_chunk(c, b):
        buf = rows_v.at[b]
        for j in range(K):
            pltpu.async_copy(
                table_hbm.at[idx_all.at[c * K + j]],
                buf.at[pl.ds(j * IDX_W, IDX_W)],
                sem_g,
            )
        for j in range(K):
            pltpu.make_async_copy(
                table_hbm.at[idx_all.at[c * K + j]],
                buf.at[pl.ds(j * IDX_W, IDX_W)],
                sem_g,
            ).wait()
        pltpu.async_copy(
            buf,
            out_hbm.at[pl.ds(row0 + c * CHUNK, CHUNK), pl.ds(0, EMBED_DIM)],
            sem_w[b])

    def wait_write(c, b):
        pltpu.make_async_copy(
            rows_v.at[b],
            out_hbm.at[pl.ds(row0 + c * CHUNK, CHUNK), pl.ds(0, EMBED_DIM)],
            sem_w[b],
        ).wait()

    for b in range(NBUF):
        do_chunk(b, b)

    @pl.loop(NBUF, NCH, step=NBUF)
    def _(cc):
        for b in range(NBUF):
            wait_write(cc + b - NBUF, b)
            do_chunk(cc + b, b)

    for b in range(NBUF):
        wait_write(NCH - NBUF + b, b)


def kernel(input, input_lengths, table):
    idx = jnp.asarray(input, jnp.int32).reshape(TOT // IDX_W, IDX_W)
    out = _embed_gather(table, idx)
    return (out[:, :EMBED_DIM].reshape(BATCH, SEQ, EMBED_DIM), input_lengths)


# speedup vs baseline: 7.4965x; 1.0616x over previous
import functools
import jax
import jax.numpy as jnp
from jax import lax
from jax.experimental import pallas as pl
from jax.experimental.pallas import tpu as pltpu
from jax.experimental.pallas import tpu_sc as plsc

VOCAB = 100000
EMBED_DIM = 64
BATCH = 4096
SEQ = 200

PAD_DIM = 128
TOT = BATCH * SEQ
NW = 32
PER_W = TOT // NW
IDX_W = 128
IDX_ROWS = PER_W // IDX_W
CHUNK = 512
K = CHUNK // IDX_W
NCH = PER_W // CHUNK
NBUF = 2

_mesh = plsc.VectorSubcoreMesh(core_axis_name="c", subcore_axis_name="s")


@functools.partial(
    pl.kernel,
    mesh=_mesh,
    out_type=jax.ShapeDtypeStruct((TOT, PAD_DIM), jnp.float32),
    compiler_params=pltpu.CompilerParams(use_tc_tiling_on_sc=False),
    scratch_types=[
        pltpu.VMEM((IDX_ROWS, IDX_W), jnp.int32),
        pltpu.VMEM((NBUF, CHUNK, EMBED_DIM), jnp.float32),
        pltpu.SemaphoreType.DMA,
        pltpu.SemaphoreType.DMA,
        pltpu.SemaphoreType.DMA,
    ],
)
def _embed_gather(table_hbm, idx_hbm, out_hbm, idx_all, rows_v, sem_g,
                  sem_w0, sem_w1):
    wid = lax.axis_index("s") * 2 + lax.axis_index("c")
    row0 = wid * PER_W
    sem_w = (sem_w0, sem_w1)

    pltpu.sync_copy(idx_hbm.at[pl.ds(wid * IDX_ROWS, IDX_ROWS)], idx_all)

    def do_chunk(c, b):
        buf = rows_v.at[b]
        for j in range(K):
            pltpu.async_copy(
                table_hbm.at[idx_all.at[c * K + j]],
                buf.at[pl.ds(j * IDX_W, IDX_W)],
                sem_g,
            )
        for j in range(K):
            pltpu.make_async_copy(
                table_hbm.at[idx_all.at[c * K + j]],
                buf.at[pl.ds(j * IDX_W, IDX_W)],
                sem_g,
            ).wait()
        pltpu.async_copy(
            buf,
            out_hbm.at[pl.ds(row0 + c * CHUNK, CHUNK), pl.ds(0, EMBED_DIM)],
            sem_w[b])

    def wait_write(c, b):
        pltpu.make_async_copy(
            rows_v.at[b],
            out_hbm.at[pl.ds(row0 + c * CHUNK, CHUNK), pl.ds(0, EMBED_DIM)],
            sem_w[b],
        ).wait()

    for b in range(NBUF):
        do_chunk(b, b)

    @pl.loop(NBUF, NCH, step=NBUF)
    def _(cc):
        for b in range(NBUF):
            wait_write(cc + b - NBUF, b)
            do_chunk(cc + b, b)

    for b in range(NBUF):
        wait_write(NCH - NBUF + b, b)


def kernel(input, input_lengths, table):
    idx = jnp.asarray(input, jnp.int32).reshape(TOT // IDX_W, IDX_W)
    out = _embed_gather(table, idx)
    return (out[:, :EMBED_DIM].reshape(BATCH, SEQ, EMBED_DIM), input_lengths)


# speedup vs baseline: 7.4979x; 1.0002x over previous
import functools
import jax
import jax.numpy as jnp
from jax import lax
from jax.experimental import pallas as pl
from jax.experimental.pallas import tpu as pltpu
from jax.experimental.pallas import tpu_sc as plsc

VOCAB = 100000
EMBED_DIM = 64
BATCH = 4096
SEQ = 200

PAD_DIM = 128
TOT = BATCH * SEQ
NW = 32
PER_W = TOT // NW
IDX_W = 128
IDX_ROWS = PER_W // IDX_W
CHUNK = 640
K = CHUNK // IDX_W
NCH = PER_W // CHUNK
NBUF = 2

_mesh = plsc.VectorSubcoreMesh(core_axis_name="c", subcore_axis_name="s")


@functools.partial(
    pl.kernel,
    mesh=_mesh,
    out_type=jax.ShapeDtypeStruct((TOT, PAD_DIM), jnp.float32),
    compiler_params=pltpu.CompilerParams(use_tc_tiling_on_sc=False),
    scratch_types=[
        pltpu.VMEM((IDX_ROWS, IDX_W), jnp.int32),
        pltpu.VMEM((NBUF, CHUNK, EMBED_DIM), jnp.float32),
        pltpu.SemaphoreType.DMA,
        pltpu.SemaphoreType.DMA,
        pltpu.SemaphoreType.DMA,
    ],
)
def _embed_gather(table_hbm, idx_hbm, out_hbm, idx_all, rows_v, sem_g,
                  sem_w0, sem_w1):
    wid = lax.axis_index("s") * 2 + lax.axis_index("c")
    row0 = wid * PER_W
    sem_w = (sem_w0, sem_w1)

    pltpu.sync_copy(idx_hbm.at[pl.ds(wid * IDX_ROWS, IDX_ROWS)], idx_all)

    def do_chunk(c, b):
        buf = rows_v.at[b]
        for j in range(K):
            pltpu.async_copy(
                table_hbm.at[idx_all.at[c * K + j]],
                buf.at[pl.ds(j * IDX_W, IDX_W)],
                sem_g,
            )
        for j in range(K):
            pltpu.make_async_copy(
                table_hbm.at[idx_all.at[c * K + j]],
                buf.at[pl.ds(j * IDX_W, IDX_W)],
                sem_g,
            ).wait()
        pltpu.async_copy(
            buf,
            out_hbm.at[pl.ds(row0 + c * CHUNK, CHUNK), pl.ds(0, EMBED_DIM)],
            sem_w[b])

    def wait_write(c, b):
        pltpu.make_async_copy(
            rows_v.at[b],
            out_hbm.at[pl.ds(row0 + c * CHUNK, CHUNK), pl.ds(0, EMBED_DIM)],
            sem_w[b],
        ).wait()

    for b in range(NBUF):
        do_chunk(b, b)

    @pl.loop(NBUF, NCH, step=NBUF)
    def _(cc):
        for b in range(NBUF):
            wait_write(cc + b - NBUF, b)
            do_chunk(cc + b, b)

    for b in range(NBUF):
        wait_write(NCH - NBUF + b, b)


def kernel(input, input_lengths, table):
    idx = jnp.asarray(input, jnp.int32).reshape(TOT // IDX_W, IDX_W)
    out = _embed_gather(table, idx)
    return (out[:, :EMBED_DIM].reshape(BATCH, SEQ, EMBED_DIM), input_lengths)
